# speedup vs baseline: 60.9121x; 60.9121x over previous
import functools
import math

import jax
import jax.numpy as jnp
from jax.experimental import pallas as pl
from jax.experimental.pallas import tpu as pltpu

_BT = 256


def _gelu_exact(h):
    return 0.5 * h * (1.0 + jax.lax.erf(h * (1.0 / math.sqrt(2.0))))


def _moe_body(x_ref, wgr_ref, wg_ref, w1_ref, b1_ref, w2_ref, b2_ref,
              o_ref, idx_ref, sc_ref, *, n_experts):
    e = pl.program_id(0)
    ti = pl.program_id(1)
    rows = pl.ds(ti * _BT, _BT)

    @pl.when(e == 0)
    def _gate():
        xb = x_ref[rows, :]
        g = jax.lax.dot_general(
            xb, wgr_ref[...], (((1,), (1,)), ((), ())),
            precision=jax.lax.Precision.HIGHEST,
            preferred_element_type=jnp.float32)
        wgn = wg_ref[...]
        norm = jnp.sqrt(jnp.sum(wgn * wgn, axis=1, keepdims=True))
        wgn = wgn / jnp.maximum(norm, 1e-4)
        logits = jax.lax.dot_general(
            g, wgn, (((1,), (1,)), ((), ())),
            precision=jax.lax.Precision.HIGHEST,
            preferred_element_type=jnp.float32)
        m = jnp.max(logits, axis=1, keepdims=True)
        s = jnp.sum(jnp.exp(logits - m), axis=1, keepdims=True)
        sc_ref[rows, :] = 1.0 / s
        lane = jax.lax.broadcasted_iota(jnp.int32, logits.shape, 1)
        idx_ref[rows, :] = jnp.min(
            jnp.where(logits == m, lane, n_experts), axis=1, keepdims=True)

    xb = x_ref[rows, :].astype(jnp.bfloat16)
    h = jnp.dot(xb, w1_ref[0], preferred_element_type=jnp.float32)
    h = _gelu_exact(h + b1_ref[0])
    y = jnp.dot(h.astype(jnp.bfloat16), w2_ref[0],
                preferred_element_type=jnp.float32)
    y = y + b2_ref[0]
    mask = idx_ref[rows, :] == e
    contrib = jnp.where(mask, y * sc_ref[rows, :], 0.0)

    @pl.when(e == 0)
    def _init():
        o_ref[rows, :] = contrib

    @pl.when(e > 0)
    def _acc():
        o_ref[rows, :] += contrib


def kernel(hidden_states, wg_red, wg, weight1, bias1, weight2, bias2):
    b, t, d = hidden_states.shape
    n_experts, _, hdim = weight1.shape
    x = hidden_states.reshape(t, d)
    nt = t // _BT

    w1b = weight1.astype(jnp.bfloat16)
    w2b = weight2.astype(jnp.bfloat16)

    out = pl.pallas_call(
        functools.partial(_moe_body, n_experts=n_experts),
        grid=(n_experts, nt),
        in_specs=[
            pl.BlockSpec((t, d), lambda e, ti: (0, 0)),
            pl.BlockSpec(wg_red.shape, lambda e, ti: (0, 0)),
            pl.BlockSpec(wg.shape, lambda e, ti: (0, 0)),
            pl.BlockSpec((1, d, hdim), lambda e, ti: (e, 0, 0)),
            pl.BlockSpec((1, 1, hdim), lambda e, ti: (e, 0, 0)),
            pl.BlockSpec((1, hdim, d), lambda e, ti: (e, 0, 0)),
            pl.BlockSpec((1, 1, d), lambda e, ti: (e, 0, 0)),
        ],
        out_specs=pl.BlockSpec((t, d), lambda e, ti: (0, 0)),
        out_shape=jax.ShapeDtypeStruct((t, d), jnp.float32),
        scratch_shapes=[
            pltpu.VMEM((t, 1), jnp.int32),
            pltpu.VMEM((t, 1), jnp.float32),
        ],
        compiler_params=pltpu.CompilerParams(
            dimension_semantics=("arbitrary", "arbitrary"),
        ),
    )(x, wg_red, wg, w1b, bias1.reshape(n_experts, 1, hdim),
      w2b, bias2.reshape(n_experts, 1, d))
    return out.reshape(b, t, d)


# speedup vs baseline: 80.5060x; 1.3217x over previous
import functools
import math

import jax
import jax.numpy as jnp
from jax import lax
from jax.experimental import pallas as pl
from jax.experimental.pallas import tpu as pltpu
from jax.experimental.pallas import tpu_sc as plsc

_BTA = 256
_BT = 256
_NHC = 2


def _gelu_exact(h):
    return 0.5 * h * (1.0 + lax.erf(h * (1.0 / math.sqrt(2.0))))


def _gate_body(x_ref, wgr_ref, wg_ref, pos_ref, ss_ref, offs_ref,
               idx_s, rank_s, sc_s, cnt_s, *, n_experts, n_tiles):
    ph = pl.program_id(0)
    i = pl.program_id(1)
    rows = pl.ds(i * _BTA, _BTA)

    @pl.when(ph == 0)
    def _phase0():
        @pl.when(i == 0)
        def _init():
            cnt_s[...] = jnp.zeros((1, 16), jnp.float32)

        xb = x_ref[...]
        g = lax.dot_general(
            xb, wgr_ref[...], (((1,), (1,)), ((), ())),
            precision=lax.Precision.HIGHEST,
            preferred_element_type=jnp.float32)
        wgn = wg_ref[...]
        norm = jnp.sqrt(jnp.sum(wgn * wgn, axis=1, keepdims=True))
        wgn = wgn / jnp.maximum(norm, 1e-4)
        logits = lax.dot_general(
            g, wgn, (((1,), (1,)), ((), ())),
            precision=lax.Precision.HIGHEST,
            preferred_element_type=jnp.float32)
        m = jnp.max(logits, axis=1, keepdims=True)
        s = jnp.sum(jnp.exp(logits - m), axis=1, keepdims=True)
        lane_e = lax.broadcasted_iota(jnp.int32, logits.shape, 1)
        idx = jnp.min(jnp.where(logits == m, lane_e, n_experts),
                      axis=1, keepdims=True)
        idx_s[rows, :] = idx
        sc_s[rows, :] = 1.0 / s

        lane16 = lax.broadcasted_iota(jnp.int32, (_BTA, 16), 1)
        onehot = (lane16 == idx).astype(jnp.float32)
        r_io = lax.broadcasted_iota(jnp.int32, (_BTA, _BTA), 0)
        c_io = lax.broadcasted_iota(jnp.int32, (_BTA, _BTA), 1)
        tri = (r_io > c_io).astype(jnp.bfloat16)
        ranks = lax.dot_general(
            tri, onehot.astype(jnp.bfloat16), (((1,), (0,)), ((), ())),
            preferred_element_type=jnp.float32)
        ranks = ranks + cnt_s[...]
        rank_s[rows, :] = jnp.sum(ranks * onehot, axis=1, keepdims=True)
        cnt_s[...] += jnp.sum(onehot, axis=0, keepdims=True)

    @pl.when(ph == 1)
    def _phase1():
        cnt = cnt_s[...]
        a_io = lax.broadcasted_iota(jnp.int32, (16, 16), 0)
        b_io = lax.broadcasted_iota(jnp.int32, (16, 16), 1)
        up = (a_io < b_io).astype(jnp.float32)
        offs = lax.dot_general(
            cnt, up, (((1,), (0,)), ((), ())),
            precision=lax.Precision.HIGHEST,
            preferred_element_type=jnp.float32)
        offs_ref[...] = offs.astype(jnp.int32)

        idx = idx_s[rows, :]
        lane16 = lax.broadcasted_iota(jnp.int32, (_BTA, 16), 1)
        onehot = (lane16 == idx).astype(jnp.float32)
        og = jnp.sum(onehot * offs, axis=1, keepdims=True)
        pos_ref[...] = (rank_s[rows, :] + og).astype(jnp.int32)
        ss_ref[...] = jnp.broadcast_to(sc_s[rows, :], (_BTA, 128))


def _gate_positions(x, wg_red, wg, n_experts):
    t, d = x.shape
    n_tiles = t // _BTA
    tile_map = lambda p, i: (jnp.where(p == 0, n_tiles, i), 0)
    pos_full, ss_full, offs = pl.pallas_call(
        functools.partial(_gate_body, n_experts=n_experts, n_tiles=n_tiles),
        grid=(2, n_tiles),
        in_specs=[
            pl.BlockSpec((_BTA, d), lambda p, i: (i, 0)),
            pl.BlockSpec(wg_red.shape, lambda p, i: (0, 0)),
            pl.BlockSpec(wg.shape, lambda p, i: (0, 0)),
        ],
        out_specs=[
            pl.BlockSpec((_BTA, 1), tile_map),
            pl.BlockSpec((_BTA, 128), tile_map),
            pl.BlockSpec((1, 16), lambda p, i: (0, 0)),
        ],
        out_shape=[
            jax.ShapeDtypeStruct((t + _BTA, 1), jnp.int32),
            jax.ShapeDtypeStruct((t + _BTA, 128), jnp.float32),
            jax.ShapeDtypeStruct((1, 16), jnp.int32),
        ],
        scratch_shapes=[
            pltpu.VMEM((t, 1), jnp.int32),
            pltpu.VMEM((t, 1), jnp.float32),
            pltpu.VMEM((t, 1), jnp.float32),
            pltpu.VMEM((1, 16), jnp.float32),
        ],
        compiler_params=pltpu.CompilerParams(
            dimension_semantics=("arbitrary", "arbitrary"),
        ),
    )(x, wg_red, wg)
    return pos_full[:t], ss_full[:t], offs


def _sc_scatter(x, ss16, pos):
    t, d = x.shape
    mesh = plsc.VectorSubcoreMesh(core_axis_name="c", subcore_axis_name="s")
    nw = 32
    b = t // nw

    @functools.partial(
        pl.kernel, mesh=mesh,
        out_type=[jax.ShapeDtypeStruct((t, d), jnp.float32),
                  jax.ShapeDtypeStruct((t, 128), jnp.float32)],
        scratch_types=[
            pltpu.VMEM((b,), jnp.int32),
            pltpu.VMEM((b, d), jnp.float32),
            pltpu.VMEM((b, 128), jnp.float32),
            pltpu.SemaphoreType.DMA,
        ],
    )
    def k(x_hbm, ss_hbm, pos_hbm, xs_hbm, sss_hbm, idx_v, rows_v, sc_v, sem):
        wid = lax.axis_index("s") * 2 + lax.axis_index("c")
        base = wid * b
        pltpu.sync_copy(pos_hbm.at[pl.ds(base, b)], idx_v)
        pltpu.sync_copy(x_hbm.at[pl.ds(base, b)], rows_v)
        pltpu.sync_copy(ss_hbm.at[pl.ds(base, b)], sc_v)
        pltpu.async_copy(rows_v, xs_hbm.at[idx_v], sem).wait()
        pltpu.async_copy(sc_v, sss_hbm.at[idx_v], sem).wait()

    return k(x, ss16, pos)


def _ffn_body(xs_ref, ss_ref, offs_ref, w1_ref, b1_ref, w2_ref, b2_ref,
              o_ref, *, n_experts):
    e = pl.program_id(0)
    hc = pl.program_id(1)
    j = pl.program_id(2)
    base = j * _BT
    rows = pl.ds(base, _BT)

    o_lo = offs_ref[0, e]
    o_hi = offs_ref[0, e + 1]
    lo = jnp.clip(o_lo - base, 0, _BT)
    hi = jnp.clip(o_hi - base, 0, _BT)

    @pl.when(jnp.logical_and(e == 0, hc == 0))
    def _zero():
        o_ref[rows, :] = jnp.zeros_like(o_ref[rows, :])

    @pl.when(hi > lo)
    def _compute():
        xb = xs_ref[...].astype(jnp.bfloat16)
        w1 = w1_ref[0].astype(jnp.bfloat16)
        h = jnp.dot(xb, w1, preferred_element_type=jnp.float32)
        h = _gelu_exact(h + b1_ref[0])
        part = jnp.dot(h.astype(jnp.bfloat16), w2_ref[0].astype(jnp.bfloat16),
                       preferred_element_type=jnp.float32)
        part = part + jnp.where(hc == 0, 1.0, 0.0) * b2_ref[0]
        y = part * ss_ref[rows, :1]
        r_io = lax.broadcasted_iota(jnp.int32, (_BT, 1), 0)
        mask = jnp.logical_and(r_io >= lo, r_io < hi)
        o_ref[rows, :] += jnp.where(mask, y, 0.0)


def _grouped_ffn(xs, sss, offs, weight1, bias1, weight2, bias2):
    t, d = xs.shape
    n_experts, _, hdim = weight1.shape
    hc = hdim // _NHC
    nt = t // _BT
    return pl.pallas_call(
        functools.partial(_ffn_body, n_experts=n_experts),
        grid=(n_experts, _NHC, nt),
        in_specs=[
            pl.BlockSpec((_BT, d), lambda e, c, j: (j, 0)),
            pl.BlockSpec((t, 128), lambda e, c, j: (0, 0)),
            pl.BlockSpec(memory_space=pltpu.SMEM),
            pl.BlockSpec((1, d, hc), lambda e, c, j: (e, 0, c)),
            pl.BlockSpec((1, 1, hc), lambda e, c, j: (e, 0, c)),
            pl.BlockSpec((1, hc, d), lambda e, c, j: (e, c, 0)),
            pl.BlockSpec((1, 1, d), lambda e, c, j: (e, 0, 0)),
        ],
        out_specs=pl.BlockSpec((t, d), lambda e, c, j: (0, 0)),
        out_shape=jax.ShapeDtypeStruct((t, d), jnp.float32),
        compiler_params=pltpu.CompilerParams(
            dimension_semantics=("arbitrary", "arbitrary", "arbitrary"),
        ),
    )(xs, sss, offs, weight1, bias1.reshape(n_experts, 1, hdim),
      weight2, bias2.reshape(n_experts, 1, d))


def _sc_gather(ys, pos):
    t, d = ys.shape
    mesh = plsc.VectorSubcoreMesh(core_axis_name="c", subcore_axis_name="s")
    nw = 32
    b = t // nw

    @functools.partial(
        pl.kernel, mesh=mesh,
        out_type=jax.ShapeDtypeStruct((t, d), jnp.float32),
        scratch_types=[
            pltpu.VMEM((b,), jnp.int32),
            pltpu.VMEM((b, d), jnp.float32),
            pltpu.SemaphoreType.DMA,
        ],
    )
    def k(ys_hbm, pos_hbm, y_hbm, idx_v, rows_v, sem):
        wid = lax.axis_index("s") * 2 + lax.axis_index("c")
        base = wid * b
        pltpu.sync_copy(pos_hbm.at[pl.ds(base, b)], idx_v)
        pltpu.async_copy(ys_hbm.at[idx_v], rows_v, sem).wait()
        pltpu.sync_copy(rows_v, y_hbm.at[pl.ds(base, b)])

    return k(ys, pos)


def kernel(hidden_states, wg_red, wg, weight1, bias1, weight2, bias2):
    bsz, t, d = hidden_states.shape
    n_experts = weight1.shape[0]
    x = hidden_states.reshape(t, d)

    pos, ss16, offs = _gate_positions(x, wg_red, wg, n_experts)
    pos1 = pos.reshape(t)
    xs, sss = _sc_scatter(x, ss16, pos1)
    ys = _grouped_ffn(xs, sss, offs, weight1, bias1, weight2, bias2)
    y = _sc_gather(ys, pos1)
    return y.reshape(bsz, t, d)


# speedup vs baseline: 87.2228x; 1.0834x over previous
import functools
import math

import jax
import jax.numpy as jnp
from jax import lax
from jax.experimental import pallas as pl
from jax.experimental.pallas import tpu as pltpu
from jax.experimental.pallas import tpu_sc as plsc

_BTA = 256
_BT = 256
_NHC = 2


def _gelu_exact(h):
    return 0.5 * h * (1.0 + lax.erf(h * (1.0 / math.sqrt(2.0))))


def _gate_body(x_ref, wgr_ref, wg_ref, pos_ref, ss_ref, offs_ref, rot_ref,
               idx_s, rank_s, sc_s, cnt_s, *, n_experts, n_tiles):
    ph = pl.program_id(0)
    i = pl.program_id(1)
    rows = pl.ds(i * _BTA, _BTA)

    @pl.when(ph == 0)
    def _phase0():
        @pl.when(i == 0)
        def _init():
            cnt_s[...] = jnp.zeros((1, 16), jnp.float32)

        xb = x_ref[...]
        g = lax.dot_general(
            xb, wgr_ref[...], (((1,), (1,)), ((), ())),
            preferred_element_type=jnp.float32)
        wgn = wg_ref[...]
        norm = jnp.sqrt(jnp.sum(wgn * wgn, axis=1, keepdims=True))
        wgn = wgn / jnp.maximum(norm, 1e-4)
        logits = lax.dot_general(
            g, wgn, (((1,), (1,)), ((), ())),
            preferred_element_type=jnp.float32)
        m = jnp.max(logits, axis=1, keepdims=True)
        s = jnp.sum(jnp.exp(logits - m), axis=1, keepdims=True)
        lane_e = lax.broadcasted_iota(jnp.int32, logits.shape, 1)
        idx = jnp.min(jnp.where(logits == m, lane_e, n_experts),
                      axis=1, keepdims=True)
        idx_s[rows, :] = idx
        sc_s[rows, :] = 1.0 / s

        lane16 = lax.broadcasted_iota(jnp.int32, (_BTA, 16), 1)
        onehot = (lane16 == idx).astype(jnp.float32)
        r_io = lax.broadcasted_iota(jnp.int32, (_BTA, _BTA), 0)
        c_io = lax.broadcasted_iota(jnp.int32, (_BTA, _BTA), 1)
        tri = (r_io > c_io).astype(jnp.bfloat16)
        ranks = lax.dot_general(
            tri, onehot.astype(jnp.bfloat16), (((1,), (0,)), ((), ())),
            preferred_element_type=jnp.float32)
        ranks = ranks + cnt_s[...]
        rank_s[rows, :] = jnp.sum(ranks * onehot, axis=1, keepdims=True)
        cnt_s[...] += jnp.sum(onehot, axis=0, keepdims=True)

    @pl.when(ph == 1)
    def _phase1():
        cnt = cnt_s[...]
        a_io = lax.broadcasted_iota(jnp.int32, (16, 16), 0)
        b_io = lax.broadcasted_iota(jnp.int32, (16, 16), 1)
        up = (a_io < b_io).astype(jnp.float32)
        offs = lax.dot_general(
            cnt, up, (((1,), (0,)), ((), ())),
            precision=lax.Precision.HIGHEST,
            preferred_element_type=jnp.float32)
        offs_ref[...] = offs.astype(jnp.int32)
        incl = (offs + cnt).astype(jnp.int32)
        jlast = jnp.floor_divide(incl - 1, _BT)
        rot_ref[...] = jnp.remainder(jlast + 1, n_tiles)

        idx = idx_s[rows, :]
        lane16 = lax.broadcasted_iota(jnp.int32, (_BTA, 16), 1)
        onehot = (lane16 == idx).astype(jnp.float32)
        og = jnp.sum(onehot * offs, axis=1, keepdims=True)
        pos_ref[...] = (rank_s[rows, :] + og).astype(jnp.int32)
        ss_ref[...] = jnp.broadcast_to(sc_s[rows, :], (_BTA, 128))


def _gate_positions(x, wg_red, wg, n_experts):
    t, d = x.shape
    n_tiles = t // _BTA
    tile_map = lambda p, i: (jnp.where(p == 0, n_tiles, i), 0)
    pos_full, ss_full, offs, rot = pl.pallas_call(
        functools.partial(_gate_body, n_experts=n_experts, n_tiles=n_tiles),
        grid=(2, n_tiles),
        in_specs=[
            pl.BlockSpec((_BTA, d), lambda p, i: (i, 0)),
            pl.BlockSpec(wg_red.shape, lambda p, i: (0, 0)),
            pl.BlockSpec(wg.shape, lambda p, i: (0, 0)),
        ],
        out_specs=[
            pl.BlockSpec((_BTA, 1), tile_map),
            pl.BlockSpec((_BTA, 128), tile_map),
            pl.BlockSpec((1, 16), lambda p, i: (0, 0)),
            pl.BlockSpec((1, 16), lambda p, i: (0, 0)),
        ],
        out_shape=[
            jax.ShapeDtypeStruct((t + _BTA, 1), jnp.int32),
            jax.ShapeDtypeStruct((t + _BTA, 128), jnp.float32),
            jax.ShapeDtypeStruct((1, 16), jnp.int32),
            jax.ShapeDtypeStruct((1, 16), jnp.int32),
        ],
        scratch_shapes=[
            pltpu.VMEM((t, 1), jnp.int32),
            pltpu.VMEM((t, 1), jnp.float32),
            pltpu.VMEM((t, 1), jnp.float32),
            pltpu.VMEM((1, 16), jnp.float32),
        ],
        compiler_params=pltpu.CompilerParams(
            dimension_semantics=("arbitrary", "arbitrary"),
        ),
    )(x, wg_red, wg)
    return pos_full[:t], ss_full[:t], offs, rot


def _sc_scatter(x, ss16, pos):
    t, d = x.shape
    mesh = plsc.VectorSubcoreMesh(core_axis_name="c", subcore_axis_name="s")
    nw = 32
    b = t // nw

    @functools.partial(
        pl.kernel, mesh=mesh,
        out_type=[jax.ShapeDtypeStruct((t, d), jnp.float32),
                  jax.ShapeDtypeStruct((t, 128), jnp.float32)],
        scratch_types=[
            pltpu.VMEM((b,), jnp.int32),
            pltpu.VMEM((b, d), jnp.float32),
            pltpu.VMEM((b, 128), jnp.float32),
            pltpu.SemaphoreType.DMA,
        ],
    )
    def k(x_hbm, ss_hbm, pos_hbm, xs_hbm, sss_hbm, idx_v, rows_v, sc_v, sem):
        wid = lax.axis_index("s") * 2 + lax.axis_index("c")
        base = wid * b
        pltpu.sync_copy(pos_hbm.at[pl.ds(base, b)], idx_v)
        pltpu.sync_copy(x_hbm.at[pl.ds(base, b)], rows_v)
        pltpu.sync_copy(ss_hbm.at[pl.ds(base, b)], sc_v)
        pltpu.async_copy(rows_v, xs_hbm.at[idx_v], sem).wait()
        pltpu.async_copy(sc_v, sss_hbm.at[idx_v], sem).wait()

    return k(x, ss16, pos)


def _ffn_body(rot_ref, offs_ref, xs_ref, ss_ref, w1_ref, b1_ref, w2_ref,
              b2_ref, o_ref, *, n_experts, n_tiles):
    e = pl.program_id(0)
    hc = pl.program_id(1)
    j = pl.program_id(2)
    jj = jnp.bitwise_and(j + rot_ref[e], n_tiles - 1)
    base = jj * _BT
    rows = pl.ds(base, _BT)

    o_lo = offs_ref[e]
    o_hi = offs_ref[e + 1]
    lo = jnp.clip(o_lo - base, 0, _BT)
    hi = jnp.clip(o_hi - base, 0, _BT)

    @pl.when(jnp.logical_and(e == 0, hc == 0))
    def _zero():
        o_ref[rows, :] = jnp.zeros_like(o_ref[rows, :])

    @pl.when(hi > lo)
    def _compute():
        xb = xs_ref[...].astype(jnp.bfloat16)
        w1 = w1_ref[0].astype(jnp.bfloat16)
        h = jnp.dot(xb, w1, preferred_element_type=jnp.float32)
        h = _gelu_exact(h + b1_ref[0])
        part = jnp.dot(h.astype(jnp.bfloat16), w2_ref[0].astype(jnp.bfloat16),
                       preferred_element_type=jnp.float32)
        part = part + jnp.where(hc == 0, 1.0, 0.0) * b2_ref[0]
        y = part * ss_ref[rows, :1]
        r_io = lax.broadcasted_iota(jnp.int32, (_BT, 1), 0)
        mask = jnp.logical_and(r_io >= lo, r_io < hi)
        o_ref[rows, :] += jnp.where(mask, y, 0.0)


def _grouped_ffn(xs, sss, offs, rot, weight1, bias1, weight2, bias2):
    t, d = xs.shape
    n_experts, _, hdim = weight1.shape
    hc = hdim // _NHC
    nt = t // _BT

    def xs_map(e, c, j, rot, offs):
        return (jnp.bitwise_and(j + rot[e], nt - 1), 0)

    grid_spec = pltpu.PrefetchScalarGridSpec(
        num_scalar_prefetch=2,
        grid=(n_experts, _NHC, nt),
        in_specs=[
            pl.BlockSpec((_BT, d), xs_map),
            pl.BlockSpec((t, 128), lambda e, c, j, r, o: (0, 0)),
            pl.BlockSpec((1, d, hc), lambda e, c, j, r, o: (e, 0, c)),
            pl.BlockSpec((1, 1, hc), lambda e, c, j, r, o: (e, 0, c)),
            pl.BlockSpec((1, hc, d), lambda e, c, j, r, o: (e, c, 0)),
            pl.BlockSpec((1, 1, d), lambda e, c, j, r, o: (e, 0, 0)),
        ],
        out_specs=pl.BlockSpec((t, d), lambda e, c, j, r, o: (0, 0)),
    )
    body = functools.partial(_ffn_body, n_experts=n_experts, n_tiles=nt)
    return pl.pallas_call(
        body,
        grid_spec=grid_spec,
        out_shape=jax.ShapeDtypeStruct((t, d), jnp.float32),
        compiler_params=pltpu.CompilerParams(
            dimension_semantics=("arbitrary", "arbitrary", "arbitrary"),
        ),
    )(rot, offs, xs, sss, weight1, bias1.reshape(n_experts, 1, hdim),
      weight2, bias2.reshape(n_experts, 1, d))


def _sc_gather(ys, pos):
    t, d = ys.shape
    mesh = plsc.VectorSubcoreMesh(core_axis_name="c", subcore_axis_name="s")
    nw = 32
    b = t // nw

    @functools.partial(
        pl.kernel, mesh=mesh,
        out_type=jax.ShapeDtypeStruct((t, d), jnp.float32),
        scratch_types=[
            pltpu.VMEM((b,), jnp.int32),
            pltpu.VMEM((b, d), jnp.float32),
            pltpu.SemaphoreType.DMA,
        ],
    )
    def k(ys_hbm, pos_hbm, y_hbm, idx_v, rows_v, sem):
        wid = lax.axis_index("s") * 2 + lax.axis_index("c")
        base = wid * b
        pltpu.sync_copy(pos_hbm.at[pl.ds(base, b)], idx_v)
        pltpu.async_copy(ys_hbm.at[idx_v], rows_v, sem).wait()
        pltpu.sync_copy(rows_v, y_hbm.at[pl.ds(base, b)])

    return k(ys, pos)


def kernel(hidden_states, wg_red, wg, weight1, bias1, weight2, bias2):
    bsz, t, d = hidden_states.shape
    n_experts = weight1.shape[0]
    x = hidden_states.reshape(t, d)

    pos, ss16, offs, rot = _gate_positions(x, wg_red, wg, n_experts)
    pos1 = pos.reshape(t)
    xs, sss = _sc_scatter(x, ss16, pos1)
    ys = _grouped_ffn(xs, sss, offs.reshape(16), rot.reshape(16),
                      weight1, bias1, weight2, bias2)
    y = _sc_gather(ys, pos1)
    return y.reshape(bsz, t, d)


# speedup vs baseline: 124.0438x; 1.4221x over previous
import functools
import math

import jax
import jax.numpy as jnp
from jax import lax
from jax.experimental import pallas as pl
from jax.experimental.pallas import tpu as pltpu
from jax.experimental.pallas import tpu_sc as plsc

_BTA = 256
_BT = 256
_NHC = 2


def _gelu_exact(h):
    return 0.5 * h * (1.0 + lax.erf(h * (1.0 / math.sqrt(2.0))))


def _gate_body(x_ref, wgr_ref, wg_ref, pos_ref, ss_ref, offs_ref, rot_ref,
               idx_s, rank_s, sc_s, cnt_s, *, n_experts, n_tiles):
    ph = pl.program_id(0)
    i = pl.program_id(1)
    rows = pl.ds(i * _BTA, _BTA)

    @pl.when(ph == 0)
    def _phase0():
        @pl.when(i == 0)
        def _init():
            cnt_s[...] = jnp.zeros((1, 16), jnp.float32)

        xb = x_ref[...]
        g = lax.dot_general(
            xb, wgr_ref[...], (((1,), (1,)), ((), ())),
            preferred_element_type=jnp.float32)
        wgn = wg_ref[...]
        norm = jnp.sqrt(jnp.sum(wgn * wgn, axis=1, keepdims=True))
        wgn = wgn / jnp.maximum(norm, 1e-4)
        logits = lax.dot_general(
            g, wgn, (((1,), (1,)), ((), ())),
            preferred_element_type=jnp.float32)
        m = jnp.max(logits, axis=1, keepdims=True)
        s = jnp.sum(jnp.exp(logits - m), axis=1, keepdims=True)
        lane_e = lax.broadcasted_iota(jnp.int32, logits.shape, 1)
        idx = jnp.min(jnp.where(logits == m, lane_e, n_experts),
                      axis=1, keepdims=True)
        idx_s[rows, :] = idx
        sc_s[rows, :] = 1.0 / s

        lane16 = lax.broadcasted_iota(jnp.int32, (_BTA, 16), 1)
        onehot = (lane16 == idx).astype(jnp.float32)
        r_io = lax.broadcasted_iota(jnp.int32, (_BTA, _BTA), 0)
        c_io = lax.broadcasted_iota(jnp.int32, (_BTA, _BTA), 1)
        tri = (r_io > c_io).astype(jnp.bfloat16)
        ranks = lax.dot_general(
            tri, onehot.astype(jnp.bfloat16), (((1,), (0,)), ((), ())),
            preferred_element_type=jnp.float32)
        ranks = ranks + cnt_s[...]
        rank_s[rows, :] = jnp.sum(ranks * onehot, axis=1, keepdims=True)
        cnt_s[...] += jnp.sum(onehot, axis=0, keepdims=True)

    @pl.when(ph == 1)
    def _phase1():
        cnt = cnt_s[...]
        a_io = lax.broadcasted_iota(jnp.int32, (16, 16), 0)
        b_io = lax.broadcasted_iota(jnp.int32, (16, 16), 1)
        up = (a_io < b_io).astype(jnp.float32)
        offs = lax.dot_general(
            cnt, up, (((1,), (0,)), ((), ())),
            precision=lax.Precision.HIGHEST,
            preferred_element_type=jnp.float32)
        offs_ref[...] = offs.astype(jnp.int32)
        incl = (offs + cnt).astype(jnp.int32)
        jlast = jnp.floor_divide(incl - 1, _BT)
        rot_ref[...] = jnp.remainder(jlast + 1, n_tiles)

        idx = idx_s[rows, :]
        lane16 = lax.broadcasted_iota(jnp.int32, (_BTA, 16), 1)
        onehot = (lane16 == idx).astype(jnp.float32)
        og = jnp.sum(onehot * offs, axis=1, keepdims=True)
        pos_ref[...] = (rank_s[rows, :] + og).astype(jnp.int32)
        ss_ref[...] = jnp.broadcast_to(sc_s[rows, :], (_BTA, 128))


def _gate_positions(x, wg_red, wg, n_experts):
    t, d = x.shape
    n_tiles = t // _BTA
    tile_map = lambda p, i: (jnp.where(p == 0, n_tiles, i), 0)
    pos_full, ss_full, offs, rot = pl.pallas_call(
        functools.partial(_gate_body, n_experts=n_experts, n_tiles=n_tiles),
        grid=(2, n_tiles),
        in_specs=[
            pl.BlockSpec((_BTA, d), lambda p, i: (i, 0)),
            pl.BlockSpec(wg_red.shape, lambda p, i: (0, 0)),
            pl.BlockSpec(wg.shape, lambda p, i: (0, 0)),
        ],
        out_specs=[
            pl.BlockSpec((_BTA, 1), tile_map),
            pl.BlockSpec((_BTA, 128), tile_map),
            pl.BlockSpec((1, 16), lambda p, i: (0, 0)),
            pl.BlockSpec((1, 16), lambda p, i: (0, 0)),
        ],
        out_shape=[
            jax.ShapeDtypeStruct((t + _BTA, 1), jnp.int32),
            jax.ShapeDtypeStruct((t + _BTA, 128), jnp.float32),
            jax.ShapeDtypeStruct((1, 16), jnp.int32),
            jax.ShapeDtypeStruct((1, 16), jnp.int32),
        ],
        scratch_shapes=[
            pltpu.VMEM((t, 1), jnp.int32),
            pltpu.VMEM((t, 1), jnp.float32),
            pltpu.VMEM((t, 1), jnp.float32),
            pltpu.VMEM((1, 16), jnp.float32),
        ],
        compiler_params=pltpu.CompilerParams(
            dimension_semantics=("arbitrary", "arbitrary"),
        ),
    )(x, wg_red, wg)
    return pos_full[:t], ss_full[:t], offs, rot


def _sc_scatter(x, ss16, pos):
    t, d = x.shape
    mesh = plsc.VectorSubcoreMesh(core_axis_name="c", subcore_axis_name="s")
    nw = 32
    b = t // nw

    @functools.partial(
        pl.kernel, mesh=mesh,
        out_type=[jax.ShapeDtypeStruct((t, d), jnp.float32),
                  jax.ShapeDtypeStruct((t, 128), jnp.float32)],
        scratch_types=[
            pltpu.VMEM((b,), jnp.int32),
            pltpu.VMEM((b, d), jnp.float32),
            pltpu.VMEM((b, 128), jnp.float32),
            pltpu.SemaphoreType.DMA,
        ],
    )
    def k(x_hbm, ss_hbm, pos_hbm, xs_hbm, sss_hbm, idx_v, rows_v, sc_v, sem):
        wid = lax.axis_index("s") * 2 + lax.axis_index("c")
        base = wid * b
        pltpu.sync_copy(pos_hbm.at[pl.ds(base, b)], idx_v)
        pltpu.sync_copy(x_hbm.at[pl.ds(base, b)], rows_v)
        pltpu.sync_copy(ss_hbm.at[pl.ds(base, b)], sc_v)
        pltpu.async_copy(rows_v, xs_hbm.at[idx_v], sem).wait()
        pltpu.async_copy(sc_v, sss_hbm.at[idx_v], sem).wait()

    return k(x, ss16, pos)


def _ffn_body(rot_ref, offs_ref, xs_ref, ss_ref, w1_ref, b1_ref, w2_ref,
              b2_ref, o_ref, *, n_experts, n_tiles):
    e = pl.program_id(0)
    hc = pl.program_id(1)
    j = pl.program_id(2)
    jj = jnp.bitwise_and(j + rot_ref[e], n_tiles - 1)
    base = jj * _BT
    rows = pl.ds(base, _BT)

    o_lo = offs_ref[e]
    o_hi = offs_ref[e + 1]
    lo = jnp.clip(o_lo - base, 0, _BT)
    hi = jnp.clip(o_hi - base, 0, _BT)

    @pl.when(jnp.logical_and(e == 0, hc == 0))
    def _zero():
        o_ref[rows, :] = jnp.zeros_like(o_ref[rows, :])

    @pl.when(hi > lo)
    def _compute():
        xb = xs_ref[rows, :].astype(jnp.bfloat16)
        w1 = w1_ref[0].astype(jnp.bfloat16)
        h = jnp.dot(xb, w1, preferred_element_type=jnp.float32)
        h = _gelu_exact(h + b1_ref[0])
        part = jnp.dot(h.astype(jnp.bfloat16), w2_ref[0].astype(jnp.bfloat16),
                       preferred_element_type=jnp.float32)
        part = part + jnp.where(hc == 0, 1.0, 0.0) * b2_ref[0]
        y = part * ss_ref[rows, :1]
        r_io = lax.broadcasted_iota(jnp.int32, (_BT, 1), 0)
        mask = jnp.logical_and(r_io >= lo, r_io < hi)
        o_ref[rows, :] += jnp.where(mask, y, 0.0)


def _grouped_ffn(xs, sss, offs, rot, weight1, bias1, weight2, bias2):
    t, d = xs.shape
    n_experts, _, hdim = weight1.shape
    hc = hdim // _NHC
    nt = t // _BT

    grid_spec = pltpu.PrefetchScalarGridSpec(
        num_scalar_prefetch=2,
        grid=(n_experts, _NHC, nt),
        in_specs=[
            pl.BlockSpec((t, d), lambda e, c, j, r, o: (0, 0)),
            pl.BlockSpec((t, 128), lambda e, c, j, r, o: (0, 0)),
            pl.BlockSpec((1, d, hc), lambda e, c, j, r, o: (e, 0, c)),
            pl.BlockSpec((1, 1, hc), lambda e, c, j, r, o: (e, 0, c)),
            pl.BlockSpec((1, hc, d), lambda e, c, j, r, o: (e, c, 0)),
            pl.BlockSpec((1, 1, d), lambda e, c, j, r, o: (e, 0, 0)),
        ],
        out_specs=pl.BlockSpec((t, d), lambda e, c, j, r, o: (0, 0)),
    )
    body = functools.partial(_ffn_body, n_experts=n_experts, n_tiles=nt)
    return pl.pallas_call(
        body,
        grid_spec=grid_spec,
        out_shape=jax.ShapeDtypeStruct((t, d), jnp.float32),
        compiler_params=pltpu.CompilerParams(
            dimension_semantics=("arbitrary", "arbitrary", "arbitrary"),
        ),
    )(rot, offs, xs, sss, weight1, bias1.reshape(n_experts, 1, hdim),
      weight2, bias2.reshape(n_experts, 1, d))


def _sc_gather(ys, pos):
    t, d = ys.shape
    mesh = plsc.VectorSubcoreMesh(core_axis_name="c", subcore_axis_name="s")
    nw = 32
    b = t // nw

    @functools.partial(
        pl.kernel, mesh=mesh,
        out_type=jax.ShapeDtypeStruct((t, d), jnp.float32),
        scratch_types=[
            pltpu.VMEM((b,), jnp.int32),
            pltpu.VMEM((b, d), jnp.float32),
            pltpu.SemaphoreType.DMA,
        ],
    )
    def k(ys_hbm, pos_hbm, y_hbm, idx_v, rows_v, sem):
        wid = lax.axis_index("s") * 2 + lax.axis_index("c")
        base = wid * b
        pltpu.sync_copy(pos_hbm.at[pl.ds(base, b)], idx_v)
        pltpu.async_copy(ys_hbm.at[idx_v], rows_v, sem).wait()
        pltpu.sync_copy(rows_v, y_hbm.at[pl.ds(base, b)])

    return k(ys, pos)


def kernel(hidden_states, wg_red, wg, weight1, bias1, weight2, bias2):
    bsz, t, d = hidden_states.shape
    n_experts = weight1.shape[0]
    x = hidden_states.reshape(t, d)

    pos, ss16, offs, rot = _gate_positions(x, wg_red, wg, n_experts)
    pos1 = pos.reshape(t)
    xs, sss = _sc_scatter(x, ss16, pos1)
    ys = _grouped_ffn(xs, sss, offs.reshape(16), rot.reshape(16),
                      weight1, bias1, weight2, bias2)
    y = _sc_gather(ys, pos1)
    return y.reshape(bsz, t, d)
